# baseline (speedup 1.0000x reference)
import functools

import jax
import jax.numpy as jnp
from jax import lax
from jax.experimental import pallas as pl
from jax.experimental.pallas import tpu as pltpu
from jax.experimental.pallas import tpu_sc as plsc

NUM_EMBEDDINGS = 1024
EMBEDDING_DIM = 64
COMMITMENT_COST = 0.25
N_ROWS = 8192
BATCH_PER_STEP = 2
BLOCK_ROWS = 1024 * BATCH_PER_STEP
N_BLOCKS = N_ROWS // BLOCK_ROWS

NC = 2
NS = 16
NW = NC * NS
ROWS_PER_W = N_ROWS // NW
CHUNK_ROWS = 64
N_CHUNKS = ROWS_PER_W // CHUNK_ROWS
CHUNK_WORDS = CHUNK_ROWS * NUM_EMBEDDINGS


def _vq_tc_kernel(x_ref, emb_ref, idx_ref, q_ref, loss_ref, perp_ref,
                  loss_acc, hist_acc):
    step = pl.program_id(0)
    x_cf = jnp.concatenate([x_ref[i] for i in range(BATCH_PER_STEP)], axis=1)
    emb = emb_ref[:]

    x2 = jnp.sum(x_cf * x_cf, axis=0, keepdims=True)
    e2 = jnp.sum(emb * emb, axis=1)[:, None]
    m = jax.lax.dot_general(emb, x_cf, (((1,), (0,)), ((), ())),
                            preferred_element_type=jnp.float32)
    d = (x2 + e2) - 2.0 * m

    idx = jnp.argmin(d, axis=0)
    idx_ref[:] = idx[None, None, :]
    idx_col = idx[:, None]
    iota = jax.lax.broadcasted_iota(jnp.int32, (BLOCK_ROWS, NUM_EMBEDDINGS), 1)
    onehot = (idx_col == iota).astype(jnp.float32)

    q = jax.lax.dot_general(emb, onehot, (((0,), (1,)), ((), ())),
                            preferred_element_type=jnp.float32)
    qst = x_cf + (q - x_cf)
    for i in range(BATCH_PER_STEP):
        q_ref[i] = qst[:, i * 1024:(i + 1) * 1024]

    @pl.when(step == 0)
    def _init():
        loss_acc[:] = jnp.zeros_like(loss_acc)
        hist_acc[:] = jnp.zeros_like(hist_acc)

    loss_acc[:] += jnp.sum((q - x_cf) ** 2).reshape(1, 1)
    hist_acc[:] += jnp.sum(onehot, axis=0, keepdims=True)

    @pl.when(step == N_BLOCKS - 1)
    def _fin():
        loss_ref[:] = COMMITMENT_COST * loss_acc[:] / (N_ROWS * EMBEDDING_DIM)
        p = hist_acc[:] / float(N_ROWS)
        perp_ref[:] = jnp.exp(-jnp.sum(p * jnp.log(p + 1e-10))).reshape(1, 1)


def _enc_sc_body(idx_hbm, out_hbm, idx_v, buf):
    wid = lax.axis_index("c") * NS + lax.axis_index("s")
    row0 = wid * ROWS_PER_W

    zeros16 = jnp.zeros((16,), jnp.float32)
    ones16 = jnp.ones((16,), jnp.float32)
    lane = lax.iota(jnp.int32, 16)

    def _zero(j, carry):
        for k in range(8):
            buf[pl.ds(j * 128 + k * 16, 16)] = zeros16
        return carry
    lax.fori_loop(0, CHUNK_WORDS // 128, _zero, 0)

    def _chunk(c, carry):
        base = row0 + c * CHUNK_ROWS
        pltpu.sync_copy(idx_hbm.at[pl.ds(base, CHUNK_ROWS)], idx_v)
        for g in range(CHUNK_ROWS // 16):
            iv = idx_v[pl.ds(g * 16, 16)]
            off = (g * 16 + lane) * NUM_EMBEDDINGS + iv
            plsc.store_scatter(buf, [off], ones16)
        pltpu.sync_copy(buf, out_hbm.at[pl.ds(base * NUM_EMBEDDINGS,
                                              CHUNK_WORDS)])
        for g in range(CHUNK_ROWS // 16):
            iv = idx_v[pl.ds(g * 16, 16)]
            off = (g * 16 + lane) * NUM_EMBEDDINGS + iv
            plsc.store_scatter(buf, [off], zeros16)
        return carry
    lax.fori_loop(0, N_CHUNKS, _chunk, 0)


_enc_sc_kernel = functools.partial(
    pl.kernel,
    mesh=plsc.VectorSubcoreMesh(core_axis_name="c", subcore_axis_name="s"),
    out_type=jax.ShapeDtypeStruct((N_ROWS * NUM_EMBEDDINGS,), jnp.float32),
    scratch_types=[
        pltpu.VMEM((CHUNK_ROWS,), jnp.int32),
        pltpu.VMEM((CHUNK_WORDS,), jnp.float32),
    ],
    compiler_params=pltpu.CompilerParams(needs_layout_passes=False,
                                         skip_device_barrier=True),
)(_enc_sc_body)


@functools.partial(jax.jit, static_argnames=("interpret",))
def kernel(inputs, emb_weight, interpret=False):
    x_cf = inputs.reshape(8, EMBEDDING_DIM, 1024)

    idx, q_st, loss, perp = pl.pallas_call(
        _vq_tc_kernel,
        grid=(N_BLOCKS,),
        in_specs=[
            pl.BlockSpec((BATCH_PER_STEP, EMBEDDING_DIM, 1024),
                         lambda i: (i, 0, 0)),
            pl.BlockSpec((NUM_EMBEDDINGS, EMBEDDING_DIM), lambda i: (0, 0)),
        ],
        out_specs=[
            pl.BlockSpec((1, 1, BLOCK_ROWS), lambda i: (i, 0, 0)),
            pl.BlockSpec((BATCH_PER_STEP, EMBEDDING_DIM, 1024),
                         lambda i: (i, 0, 0)),
            pl.BlockSpec((1, 1), lambda i: (0, 0)),
            pl.BlockSpec((1, 1), lambda i: (0, 0)),
        ],
        out_shape=[
            jax.ShapeDtypeStruct((N_BLOCKS, 1, BLOCK_ROWS), jnp.int32),
            jax.ShapeDtypeStruct((8, EMBEDDING_DIM, 1024), jnp.float32),
            jax.ShapeDtypeStruct((1, 1), jnp.float32),
            jax.ShapeDtypeStruct((1, 1), jnp.float32),
        ],
        scratch_shapes=[
            pltpu.VMEM((1, 1), jnp.float32),
            pltpu.VMEM((1, NUM_EMBEDDINGS), jnp.float32),
        ],
        interpret=interpret,
    )(x_cf, emb_weight)

    enc = _enc_sc_kernel(idx.reshape(N_ROWS)).reshape(N_ROWS, NUM_EMBEDDINGS)

    quantized_out = q_st.reshape(8, EMBEDDING_DIM, 32, 32)
    return (loss[0, 0], quantized_out, perp[0, 0], enc)


# speedup vs baseline: 2.8055x; 2.8055x over previous
import functools

import jax
import jax.numpy as jnp
from jax.experimental import pallas as pl
from jax.experimental.pallas import tpu as pltpu

NUM_EMBEDDINGS = 1024
EMBEDDING_DIM = 64
COMMITMENT_COST = 0.25
N_ROWS = 8192
BATCH_PER_STEP = 2
BLOCK_ROWS = 1024 * BATCH_PER_STEP
N_BLOCKS = N_ROWS // BLOCK_ROWS


def _vq_kernel(x_ref, emb_ref, enc_ref, q_ref, loss_ref, perp_ref,
               loss_acc, hist_acc):
    step = pl.program_id(0)
    x_cf = jnp.concatenate([x_ref[i] for i in range(BATCH_PER_STEP)], axis=1)
    emb = emb_ref[:]

    x2 = jnp.sum(x_cf * x_cf, axis=0, keepdims=True)
    e2 = jnp.sum(emb * emb, axis=1)[:, None]
    m2 = jax.lax.dot_general(emb + emb, x_cf, (((1,), (0,)), ((), ())),
                             preferred_element_type=jnp.float32)
    d = (x2 + e2) - m2

    idx = jnp.argmin(d, axis=0)
    idx_col = idx[:, None]
    iota = jax.lax.broadcasted_iota(jnp.int32, (BLOCK_ROWS, NUM_EMBEDDINGS), 1)
    onehot = (idx_col == iota).astype(jnp.float32)
    enc_ref[:] = onehot

    q = jax.lax.dot_general(emb, onehot, (((0,), (1,)), ((), ())),
                            preferred_element_type=jnp.float32)
    qst = x_cf + (q - x_cf)
    for i in range(BATCH_PER_STEP):
        q_ref[i] = qst[:, i * 1024:(i + 1) * 1024]

    @pl.when(step == 0)
    def _init():
        loss_acc[:] = jnp.zeros_like(loss_acc)
        hist_acc[:] = jnp.zeros_like(hist_acc)

    loss_acc[:] += jnp.sum((q - x_cf) ** 2).reshape(1, 1)
    hist_acc[:] += jax.lax.dot_general(
        jnp.ones((1, BLOCK_ROWS), jnp.float32), onehot,
        (((1,), (0,)), ((), ())), preferred_element_type=jnp.float32)

    @pl.when(step == N_BLOCKS - 1)
    def _fin():
        loss_ref[:] = COMMITMENT_COST * loss_acc[:] / (N_ROWS * EMBEDDING_DIM)
        p = hist_acc[:] / float(N_ROWS)
        perp_ref[:] = jnp.exp(-jnp.sum(p * jnp.log(p + 1e-10))).reshape(1, 1)


@functools.partial(jax.jit, static_argnames=("interpret",))
def kernel(inputs, emb_weight, interpret=False):
    x_cf = inputs.reshape(8, EMBEDDING_DIM, 1024)

    enc, q_st, loss, perp = pl.pallas_call(
        _vq_kernel,
        grid=(N_BLOCKS,),
        in_specs=[
            pl.BlockSpec((BATCH_PER_STEP, EMBEDDING_DIM, 1024),
                         lambda i: (i, 0, 0)),
            pl.BlockSpec((NUM_EMBEDDINGS, EMBEDDING_DIM), lambda i: (0, 0)),
        ],
        out_specs=[
            pl.BlockSpec((BLOCK_ROWS, NUM_EMBEDDINGS), lambda i: (i, 0)),
            pl.BlockSpec((BATCH_PER_STEP, EMBEDDING_DIM, 1024),
                         lambda i: (i, 0, 0)),
            pl.BlockSpec((1, 1), lambda i: (0, 0)),
            pl.BlockSpec((1, 1), lambda i: (0, 0)),
        ],
        out_shape=[
            jax.ShapeDtypeStruct((N_ROWS, NUM_EMBEDDINGS), jnp.float32),
            jax.ShapeDtypeStruct((8, EMBEDDING_DIM, 1024), jnp.float32),
            jax.ShapeDtypeStruct((1, 1), jnp.float32),
            jax.ShapeDtypeStruct((1, 1), jnp.float32),
        ],
        scratch_shapes=[
            pltpu.VMEM((1, 1), jnp.float32),
            pltpu.VMEM((1, NUM_EMBEDDINGS), jnp.float32),
        ],
        interpret=interpret,
    )(x_cf, emb_weight)

    quantized_out = q_st.reshape(8, EMBEDDING_DIM, 32, 32)
    return (loss[0, 0], quantized_out, perp[0, 0], enc)
